# Initial kernel scaffold; baseline (speedup 1.0000x reference)
import jax
import jax.numpy as jnp
from jax.experimental import pallas as pl


def kernel(xyz, points, seed_inds, W1, b1, g1, bt1, W2, b2, g2, bt2, W3, b3, g3, bt3):
    raise NotImplementedError("write your pallas kernel here")


# speedup vs baseline: 7.8418x; 7.8418x over previous
import functools

import numpy as np
import jax
import jax.numpy as jnp
from jax import lax
from jax.experimental import pallas as pl
from jax.experimental.pallas import tpu as pltpu
from jax.experimental.pallas import tpu_sc as plsc

B_ = 4
N_ = 16384
D_ = 128
NPT = 1024
NS = 32
RAD2 = np.float32(0.4 ** 2)
NW = 32
NQT = NPT // NW
RW = 144
C3 = 256


def _fps_body(xyz_ref, cent_ref, coord_ref, dist_ref):
    x = xyz_ref[:, 0]
    y = xyz_ref[:, 1]
    z = xyz_ref[:, 2]
    pos = (lax.broadcasted_iota(jnp.int32, (B_, 128, 128), 1) * 128
           + lax.broadcasted_iota(jnp.int32, (B_, 128, 128), 2))
    lane = lax.broadcasted_iota(jnp.int32, (B_, 1, 128), 2)
    dist_ref[...] = jnp.full((B_, 128, 128), 1e10, jnp.float32)

    def body(i, far):
        eq = (pos == far).astype(jnp.float32)
        cx = jnp.sum(jnp.sum(x * eq, axis=2, keepdims=True), axis=1, keepdims=True)
        cy = jnp.sum(jnp.sum(y * eq, axis=2, keepdims=True), axis=1, keepdims=True)
        cz = jnp.sum(jnp.sum(z * eq, axis=2, keepdims=True), axis=1, keepdims=True)
        cent_ref[:, pl.ds(i, 1), :] = jnp.broadcast_to(far, (B_, 1, 128))
        crow = jnp.where(lane == 0, cx,
                         jnp.where(lane == 1, cy,
                                   jnp.where(lane == 2, cz, 0.0)))
        coord_ref[:, pl.ds(i, 1), :] = crow
        dx = x - cx
        dy = y - cy
        dz = z - cz
        d = (dx * dx + dy * dy) + dz * dz
        dist = jnp.minimum(dist_ref[...], d)
        dist_ref[...] = dist
        m = jnp.max(jnp.max(dist, axis=2, keepdims=True), axis=1, keepdims=True)
        cand = jnp.where(dist == m, pos, jnp.int32(2 ** 30))
        return jnp.min(jnp.min(cand, axis=2, keepdims=True), axis=1, keepdims=True)

    lax.fori_loop(0, NPT, body, jnp.zeros((B_, 1, 1), jnp.int32))


def _run_fps(xyz):
    xyz4 = xyz.reshape(B_, 3, 128, 128)
    return pl.pallas_call(
        _fps_body,
        out_shape=[jax.ShapeDtypeStruct((B_, NPT, 128), jnp.int32),
                   jax.ShapeDtypeStruct((B_, NPT, 128), jnp.float32)],
        scratch_shapes=[pltpu.VMEM((B_, 128, 128), jnp.float32)],
    )(xyz4)


def _sc_body(xyz_hbm, nxyz_hbm, fpsi_hbm, seeds_hbm, ptab_hbm,
             gx_hbm, seedo_hbm,
             xtab, ytab, ztab, ctab, qbuf, fpsb, seedtab, seedout,
             idx_all, gidx, rows, gsem):
    cid = lax.axis_index("c")
    sid = lax.axis_index("s")
    wid = sid * 2 + cid
    q0 = wid * NQT
    iota16 = lax.iota(jnp.int32, 16)

    for b in range(B_):
        pltpu.sync_copy(xyz_hbm.at[pl.ds((b * 3 + 0) * N_, N_)], xtab)
        pltpu.sync_copy(xyz_hbm.at[pl.ds((b * 3 + 1) * N_, N_)], ytab)
        pltpu.sync_copy(xyz_hbm.at[pl.ds((b * 3 + 2) * N_, N_)], ztab)
        pltpu.sync_copy(seeds_hbm.at[pl.ds(b * N_, N_)], seedtab)
        pltpu.sync_copy(nxyz_hbm.at[pl.ds((b * NPT + q0) * 4, NQT * 4)], qbuf)
        pltpu.sync_copy(fpsi_hbm.at[pl.ds(b * NPT + q0, NQT)], fpsb)

        def ct_body(i, _):
            xv = xtab[pl.ds(i * 16, 16)]
            yv = ytab[pl.ds(i * 16, 16)]
            zv = ztab[pl.ds(i * 16, 16)]
            ctab[pl.ds(i * 16, 16)] = (xv * xv + yv * yv) + zv * zv
            return 0

        lax.fori_loop(0, N_ // 16, ct_body, 0)

        for c in range(NQT // 16):
            iv = fpsb[pl.ds(c * 16, 16)]
            seedout[pl.ds(c * 16, 16)] = plsc.load_gather(seedtab, [iv])
        pltpu.sync_copy(seedout, seedo_hbm.at[pl.ds(b * NPT + q0, NQT)])

        def q_body(q, _):
            qx = plsc.load_gather(qbuf, [jnp.full((16,), q * 4 + 0, jnp.int32)])
            qy = plsc.load_gather(qbuf, [jnp.full((16,), q * 4 + 1, jnp.int32)])
            qz = plsc.load_gather(qbuf, [jnp.full((16,), q * 4 + 2, jnp.int32)])
            sq = (qx * qx + qy * qy) + qz * qz

            def cond(st):
                i, cnt = st
                return jnp.logical_and(cnt < NS, i < N_ // 16)

            def body(st):
                i, cnt = st
                base = i * 16
                xv = xtab[pl.ds(base, 16)]
                yv = ytab[pl.ds(base, 16)]
                zv = ztab[pl.ds(base, 16)]
                cv = ctab[pl.ds(base, 16)]
                m = (qx * xv + qy * yv) + qz * zv
                d = (m * jnp.float32(-2.0) + sq) + cv
                msk = d <= RAD2
                mi = jnp.where(msk, 1, 0).astype(jnp.int32)
                rank = plsc.cumsum(mi)
                keep = jnp.logical_and(msk, rank <= (NS - cnt))
                jv = base + iota16
                plsc.store_compressed(idx_all.at[pl.ds(q * NS + cnt, 16)], jv, mask=keep)
                cnt = cnt + plsc.all_reduce_population_count(keep)[0]
                return (i + 1, cnt)

            _, cnt_fin = lax.while_loop(cond, body, (jnp.int32(0), jnp.int32(0)))

            first = plsc.load_gather(idx_all, [jnp.full((16,), q * NS, jnp.int32)])
            for c in range(NS // 16):
                sl = idx_all[pl.ds(q * NS + c * 16, 16)]
                slot = iota16 + (c * 16)
                idxv = jnp.where(slot < cnt_fin, sl, first)
                gidx[pl.ds(q * NS + c * 16, 16)] = idxv + b * N_
            return 0

        lax.fori_loop(0, NQT, q_body, 0)

        row0 = (b * NPT + q0) * NS

        def g_body(t, _):
            pltpu.async_copy(ptab_hbm.at[gidx.at[pl.ds(t * 64, 64)]], rows, gsem).wait()
            pltpu.sync_copy(rows, gx_hbm.at[pl.ds(row0 + t * 64, 64)])
            return 0

        lax.fori_loop(0, NQT * NS // 64, g_body, 0)


def _run_sc(xyz, nxyz, fpsi, seed_inds, ptab):
    mesh = plsc.VectorSubcoreMesh(core_axis_name="c", subcore_axis_name="s")
    call = functools.partial(
        pl.kernel,
        mesh=mesh,
        compiler_params=pltpu.CompilerParams(needs_layout_passes=False),
        out_type=[jax.ShapeDtypeStruct((B_ * NPT * NS, D_), jnp.float32),
                  jax.ShapeDtypeStruct((B_ * NPT,), jnp.int32)],
        scratch_types=[
            pltpu.VMEM((N_,), jnp.float32),
            pltpu.VMEM((N_,), jnp.float32),
            pltpu.VMEM((N_,), jnp.float32),
            pltpu.VMEM((N_,), jnp.float32),
            pltpu.VMEM((NQT * 4,), jnp.float32),
            pltpu.VMEM((NQT,), jnp.int32),
            pltpu.VMEM((N_,), jnp.int32),
            pltpu.VMEM((NQT,), jnp.int32),
            pltpu.VMEM((NQT * NS + 24,), jnp.int32),
            pltpu.VMEM((NQT * NS,), jnp.int32),
            pltpu.VMEM((64, D_), jnp.float32),
            pltpu.SemaphoreType.DMA,
        ],
    )(_sc_body)
    return call(xyz, nxyz, fpsi, seed_inds, ptab)


_ROWS = B_ * NPT * NS
_BLK = 512


def _mm(a, b):
    return lax.dot_general(a, b, (((1,), (1,)), ((), ())),
                           precision=lax.Precision.HIGHEST,
                           preferred_element_type=jnp.float32)


def _stats_accum(s_ref, y):
    @pl.when(pl.program_id(0) == 0)
    def _():
        s_ref[...] = jnp.zeros_like(s_ref)

    s_ref[0:1, :] += jnp.sum(y, axis=0, keepdims=True)
    s_ref[1:2, :] += jnp.sum(y * y, axis=0, keepdims=True)


def _t1_body(x_ref, w_ref, b_ref, y_ref):
    y_ref[...] = _mm(x_ref[...], w_ref[...]) + b_ref[...]


def _mlp1_body(gx_ref, nx_ref, wx_ref, y_ref, s_ref):
    corr = _mm(nx_ref[...], wx_ref[...])
    corrb = jnp.broadcast_to(corr.reshape(16, 1, D_), (16, NS, D_)).reshape(_BLK, D_)
    y = gx_ref[...] - corrb
    y_ref[...] = y
    _stats_accum(s_ref, y)


def _mlp_mid_body(x_ref, sc_ref, sh_ref, w_ref, b_ref, y_ref, s_ref):
    h = jnp.maximum(x_ref[...] * sc_ref[...] + sh_ref[...], 0.0)
    y = _mm(h, w_ref[...]) + b_ref[...]
    y_ref[...] = y
    _stats_accum(s_ref, y)


def _mlp_out_body(x_ref, sc_ref, sh_ref, o_ref):
    h = jnp.maximum(x_ref[...] * sc_ref[...] + sh_ref[...], 0.0)
    o_ref[...] = jnp.max(h.reshape(32, NS, C3), axis=1)


def _run_t1(tb, w1e, b1):
    grid = B_ * N_ // _BLK
    return pl.pallas_call(
        _t1_body,
        grid=(grid,),
        in_specs=[pl.BlockSpec((_BLK, RW), lambda i: (i, 0)),
                  pl.BlockSpec((D_, RW), lambda i: (0, 0)),
                  pl.BlockSpec((1, D_), lambda i: (0, 0))],
        out_specs=pl.BlockSpec((_BLK, D_), lambda i: (i, 0)),
        out_shape=jax.ShapeDtypeStruct((B_ * N_, D_), jnp.float32),
    )(tb, w1e, b1.reshape(1, D_))


def _run_mlp1(gx, nx4, w1x4):
    grid = _ROWS // _BLK
    return pl.pallas_call(
        _mlp1_body,
        grid=(grid,),
        in_specs=[pl.BlockSpec((_BLK, D_), lambda i: (i, 0)),
                  pl.BlockSpec((16, 4), lambda i: (i, 0)),
                  pl.BlockSpec((D_, 4), lambda i: (0, 0))],
        out_specs=[pl.BlockSpec((_BLK, D_), lambda i: (i, 0)),
                   pl.BlockSpec((8, D_), lambda i: (0, 0))],
        out_shape=[jax.ShapeDtypeStruct((_ROWS, D_), jnp.float32),
                   jax.ShapeDtypeStruct((8, D_), jnp.float32)],
    )(gx, nx4, w1x4)


def _run_mlp_mid(y, scale, shift, w, b, cout):
    grid = _ROWS // _BLK
    cin = y.shape[1]
    return pl.pallas_call(
        _mlp_mid_body,
        grid=(grid,),
        in_specs=[pl.BlockSpec((_BLK, cin), lambda i: (i, 0)),
                  pl.BlockSpec((1, cin), lambda i: (0, 0)),
                  pl.BlockSpec((1, cin), lambda i: (0, 0)),
                  pl.BlockSpec((cout, cin), lambda i: (0, 0)),
                  pl.BlockSpec((1, cout), lambda i: (0, 0))],
        out_specs=[pl.BlockSpec((_BLK, cout), lambda i: (i, 0)),
                   pl.BlockSpec((8, cout), lambda i: (0, 0))],
        out_shape=[jax.ShapeDtypeStruct((_ROWS, cout), jnp.float32),
                   jax.ShapeDtypeStruct((8, cout), jnp.float32)],
    )(y, scale.reshape(1, cin), shift.reshape(1, cin), w, b.reshape(1, cout))


def _run_mlp_out(y3, scale, shift):
    grid = _ROWS // 1024
    return pl.pallas_call(
        _mlp_out_body,
        grid=(grid,),
        in_specs=[pl.BlockSpec((1024, C3), lambda i: (i, 0)),
                  pl.BlockSpec((1, C3), lambda i: (0, 0)),
                  pl.BlockSpec((1, C3), lambda i: (0, 0))],
        out_specs=pl.BlockSpec((32, C3), lambda i: (i, 0)),
        out_shape=jax.ShapeDtypeStruct((B_ * NPT, C3), jnp.float32),
    )(y3, scale.reshape(1, C3), shift.reshape(1, C3))


def _bn_fold(s, g, bt):
    rn = jnp.float32(_ROWS)
    mean = s[0] / rn
    var = s[1] / rn - mean * mean
    inv = 1.0 / jnp.sqrt(var + 1e-5)
    scale = g * inv
    shift = bt - mean * scale
    return scale, shift


def kernel(xyz, points, seed_inds, W1, b1, g1, bt1, W2, b2, g2, bt2,
           W3, b3, g3, bt3):
    cent, coord = _run_fps(xyz)
    fpsi = cent[:, :, 0]
    nxyz = coord[:, :, :4].reshape(B_, NPT * 4)

    ptst = jnp.transpose(points, (0, 2, 1))
    xyzt = jnp.transpose(xyz, (0, 2, 1))
    tb = jnp.concatenate(
        [ptst, xyzt, jnp.zeros((B_, N_, RW - D_ - 3), jnp.float32)],
        axis=2).reshape(B_ * N_, RW)
    w1e = jnp.concatenate(
        [W1[:, 3:], W1[:, :3], jnp.zeros((D_, RW - D_ - 3), jnp.float32)], axis=1)
    t1 = _run_t1(tb, w1e, b1)

    gx, seedo = _run_sc(xyz.reshape(-1), nxyz.reshape(-1), fpsi.reshape(-1),
                        seed_inds.reshape(-1), t1)
    seedo = seedo.reshape(B_, NPT)

    nx4 = coord[:, :, :4].reshape(B_ * NPT, 4)
    w1x4 = jnp.pad(W1[:, :3], ((0, 0), (0, 1)))
    y1, s1 = _run_mlp1(gx, nx4, w1x4)
    sc1, sh1 = _bn_fold(s1, g1, bt1)
    y2, s2 = _run_mlp_mid(y1, sc1, sh1, W2, b2, D_)
    sc2, sh2 = _bn_fold(s2, g2, bt2)
    y3, s3 = _run_mlp_mid(y2, sc2, sh2, W3, b3, C3)
    sc3, sh3 = _bn_fold(s3, g3, bt3)
    xo = _run_mlp_out(y3, sc3, sh3)

    new_xyz = jnp.transpose(coord[:, :, :3], (0, 2, 1))
    x_out = jnp.transpose(xo.reshape(B_, NPT, C3), (0, 2, 1))
    return (new_xyz, x_out, seedo)


# speedup vs baseline: 8.9323x; 1.1391x over previous
import functools

import numpy as np
import jax
import jax.numpy as jnp
from jax import lax
from jax.experimental import pallas as pl
from jax.experimental.pallas import tpu as pltpu
from jax.experimental.pallas import tpu_sc as plsc

B_ = 4
N_ = 16384
D_ = 128
NPT = 1024
NS = 32
RAD2 = np.float32(0.4 ** 2)
NW = 32
NQT = NPT // NW
RW = 144
C3 = 256


def _fps_body(xyz_ref, cent_ref, coord_ref, dist_ref):
    x = xyz_ref[:, 0]
    y = xyz_ref[:, 1]
    z = xyz_ref[:, 2]
    pos = (lax.broadcasted_iota(jnp.int32, (B_, 128, 128), 1) * 128
           + lax.broadcasted_iota(jnp.int32, (B_, 128, 128), 2))
    lane = lax.broadcasted_iota(jnp.int32, (B_, 1, 128), 2)
    dist_ref[...] = jnp.full((B_, 128, 128), 1e10, jnp.float32)

    def body(i, far):
        eq = (pos == far).astype(jnp.float32)
        cx = jnp.sum(jnp.sum(x * eq, axis=2, keepdims=True), axis=1, keepdims=True)
        cy = jnp.sum(jnp.sum(y * eq, axis=2, keepdims=True), axis=1, keepdims=True)
        cz = jnp.sum(jnp.sum(z * eq, axis=2, keepdims=True), axis=1, keepdims=True)
        cent_ref[:, pl.ds(i, 1), :] = jnp.broadcast_to(far, (B_, 1, 128))
        crow = jnp.where(lane == 0, cx,
                         jnp.where(lane == 1, cy,
                                   jnp.where(lane == 2, cz, 0.0)))
        coord_ref[:, pl.ds(i, 1), :] = crow
        dx = x - cx
        dy = y - cy
        dz = z - cz
        d = (dx * dx + dy * dy) + dz * dz
        dist = jnp.minimum(dist_ref[...], d)
        dist_ref[...] = dist
        m = jnp.max(jnp.max(dist, axis=2, keepdims=True), axis=1, keepdims=True)
        cand = jnp.where(dist == m, pos, jnp.int32(2 ** 30))
        return jnp.min(jnp.min(cand, axis=2, keepdims=True), axis=1, keepdims=True)

    lax.fori_loop(0, NPT, body, jnp.zeros((B_, 1, 1), jnp.int32))


def _run_fps(xyz):
    xyz4 = xyz.reshape(B_, 3, 128, 128)
    return pl.pallas_call(
        _fps_body,
        out_shape=[jax.ShapeDtypeStruct((B_, NPT, 128), jnp.int32),
                   jax.ShapeDtypeStruct((B_, NPT, 128), jnp.float32)],
        scratch_shapes=[pltpu.VMEM((B_, 128, 128), jnp.float32)],
    )(xyz4)


def _sc_body(xyz_hbm, nxyz_hbm, fpsi_hbm, seeds_hbm, ptab_hbm,
             gx_hbm, seedo_hbm,
             xtab, ytab, ztab, ctab, qbuf, fpsb, seedtab, seedout,
             idx_all, gidx, rows, gsem):
    cid = lax.axis_index("c")
    sid = lax.axis_index("s")
    wid = sid * 2 + cid
    q0 = wid * NQT
    iota16 = lax.iota(jnp.int32, 16)

    for b in range(B_):
        pltpu.sync_copy(xyz_hbm.at[pl.ds((b * 3 + 0) * N_, N_)], xtab)
        pltpu.sync_copy(xyz_hbm.at[pl.ds((b * 3 + 1) * N_, N_)], ytab)
        pltpu.sync_copy(xyz_hbm.at[pl.ds((b * 3 + 2) * N_, N_)], ztab)
        pltpu.sync_copy(seeds_hbm.at[pl.ds(b * N_, N_)], seedtab)
        pltpu.sync_copy(nxyz_hbm.at[pl.ds((b * NPT + q0) * 4, NQT * 4)], qbuf)
        pltpu.sync_copy(fpsi_hbm.at[pl.ds(b * NPT + q0, NQT)], fpsb)

        def ct_body(i, _):
            xv = xtab[pl.ds(i * 16, 16)]
            yv = ytab[pl.ds(i * 16, 16)]
            zv = ztab[pl.ds(i * 16, 16)]
            ctab[pl.ds(i * 16, 16)] = (xv * xv + yv * yv) + zv * zv
            return 0

        lax.fori_loop(0, N_ // 16, ct_body, 0)

        for c in range(NQT // 16):
            iv = fpsb[pl.ds(c * 16, 16)]
            seedout[pl.ds(c * 16, 16)] = plsc.load_gather(seedtab, [iv])
        pltpu.sync_copy(seedout, seedo_hbm.at[pl.ds(b * NPT + q0, NQT)])

        def q_body(q, _):
            qx = plsc.load_gather(qbuf, [jnp.full((16,), q * 4 + 0, jnp.int32)])
            qy = plsc.load_gather(qbuf, [jnp.full((16,), q * 4 + 1, jnp.int32)])
            qz = plsc.load_gather(qbuf, [jnp.full((16,), q * 4 + 2, jnp.int32)])
            sq = (qx * qx + qy * qy) + qz * qz

            def cond(st):
                i, cnt = st
                return jnp.logical_and(cnt < NS, i < N_ // 16)

            def body(st):
                i, cnt = st
                base = i * 16
                xv = xtab[pl.ds(base, 16)]
                yv = ytab[pl.ds(base, 16)]
                zv = ztab[pl.ds(base, 16)]
                cv = ctab[pl.ds(base, 16)]
                m = (qx * xv + qy * yv) + qz * zv
                d = (m * jnp.float32(-2.0) + sq) + cv
                msk = d <= RAD2
                jv = base + iota16
                plsc.store_compressed(idx_all.at[pl.ds(q * NS + cnt, 16)], jv, mask=msk)
                cnt = cnt + plsc.all_reduce_population_count(msk)[0]
                return (i + 1, cnt)

            _, cnt_fin = lax.while_loop(cond, body, (jnp.int32(0), jnp.int32(0)))

            first = plsc.load_gather(idx_all, [jnp.full((16,), q * NS, jnp.int32)])
            for c in range(NS // 16):
                sl = idx_all[pl.ds(q * NS + c * 16, 16)]
                slot = iota16 + (c * 16)
                idxv = jnp.where(slot < cnt_fin, sl, first)
                gidx[pl.ds(q * NS + c * 16, 16)] = idxv + b * N_
            return 0

        lax.fori_loop(0, NQT, q_body, 0)

        row0 = (b * NPT + q0) * NS

        def g_body(t, _):
            pltpu.async_copy(ptab_hbm.at[gidx.at[pl.ds(t * 64, 64)]], rows, gsem).wait()
            pltpu.sync_copy(rows, gx_hbm.at[pl.ds(row0 + t * 64, 64)])
            return 0

        lax.fori_loop(0, NQT * NS // 64, g_body, 0)


def _run_sc(xyz, nxyz, fpsi, seed_inds, ptab):
    mesh = plsc.VectorSubcoreMesh(core_axis_name="c", subcore_axis_name="s")
    call = functools.partial(
        pl.kernel,
        mesh=mesh,
        compiler_params=pltpu.CompilerParams(needs_layout_passes=False),
        out_type=[jax.ShapeDtypeStruct((B_ * NPT * NS, D_), jnp.float32),
                  jax.ShapeDtypeStruct((B_ * NPT,), jnp.int32)],
        scratch_types=[
            pltpu.VMEM((N_,), jnp.float32),
            pltpu.VMEM((N_,), jnp.float32),
            pltpu.VMEM((N_,), jnp.float32),
            pltpu.VMEM((N_,), jnp.float32),
            pltpu.VMEM((NQT * 4,), jnp.float32),
            pltpu.VMEM((NQT,), jnp.int32),
            pltpu.VMEM((N_,), jnp.int32),
            pltpu.VMEM((NQT,), jnp.int32),
            pltpu.VMEM((NQT * NS + 24,), jnp.int32),
            pltpu.VMEM((NQT * NS,), jnp.int32),
            pltpu.VMEM((64, D_), jnp.float32),
            pltpu.SemaphoreType.DMA,
        ],
    )(_sc_body)
    return call(xyz, nxyz, fpsi, seed_inds, ptab)


_ROWS = B_ * NPT * NS
_BLK = 512


def _mm(a, b):
    return lax.dot_general(a, b, (((1,), (1,)), ((), ())),
                           precision=lax.Precision.HIGHEST,
                           preferred_element_type=jnp.float32)


def _stats_accum(s_ref, y):
    @pl.when(pl.program_id(0) == 0)
    def _():
        s_ref[...] = jnp.zeros_like(s_ref)

    s_ref[0:1, :] += jnp.sum(y, axis=0, keepdims=True)
    s_ref[1:2, :] += jnp.sum(y * y, axis=0, keepdims=True)


def _t1_body(x_ref, w_ref, b_ref, y_ref):
    y_ref[...] = _mm(x_ref[...], w_ref[...]) + b_ref[...]


def _mlp1_body(gx_ref, nx_ref, wx_ref, y_ref, s_ref):
    corr = _mm(nx_ref[...], wx_ref[...])
    corrb = jnp.broadcast_to(corr.reshape(16, 1, D_), (16, NS, D_)).reshape(_BLK, D_)
    y = gx_ref[...] - corrb
    y_ref[...] = y
    _stats_accum(s_ref, y)


def _mlp_mid_body(x_ref, sc_ref, sh_ref, w_ref, b_ref, y_ref, s_ref):
    h = jnp.maximum(x_ref[...] * sc_ref[...] + sh_ref[...], 0.0)
    y = _mm(h, w_ref[...]) + b_ref[...]
    y_ref[...] = y
    _stats_accum(s_ref, y)


def _mlp_out_body(x_ref, sc_ref, sh_ref, o_ref):
    h = jnp.maximum(x_ref[...] * sc_ref[...] + sh_ref[...], 0.0)
    o_ref[...] = jnp.max(h.reshape(32, NS, C3), axis=1)


def _run_t1(tb, w1e, b1):
    grid = B_ * N_ // _BLK
    return pl.pallas_call(
        _t1_body,
        grid=(grid,),
        in_specs=[pl.BlockSpec((_BLK, RW), lambda i: (i, 0)),
                  pl.BlockSpec((D_, RW), lambda i: (0, 0)),
                  pl.BlockSpec((1, D_), lambda i: (0, 0))],
        out_specs=pl.BlockSpec((_BLK, D_), lambda i: (i, 0)),
        out_shape=jax.ShapeDtypeStruct((B_ * N_, D_), jnp.float32),
    )(tb, w1e, b1.reshape(1, D_))


def _run_mlp1(gx, nx4, w1x4):
    grid = _ROWS // _BLK
    return pl.pallas_call(
        _mlp1_body,
        grid=(grid,),
        in_specs=[pl.BlockSpec((_BLK, D_), lambda i: (i, 0)),
                  pl.BlockSpec((16, 4), lambda i: (i, 0)),
                  pl.BlockSpec((D_, 4), lambda i: (0, 0))],
        out_specs=[pl.BlockSpec((_BLK, D_), lambda i: (i, 0)),
                   pl.BlockSpec((8, D_), lambda i: (0, 0))],
        out_shape=[jax.ShapeDtypeStruct((_ROWS, D_), jnp.float32),
                   jax.ShapeDtypeStruct((8, D_), jnp.float32)],
    )(gx, nx4, w1x4)


def _run_mlp_mid(y, scale, shift, w, b, cout):
    grid = _ROWS // _BLK
    cin = y.shape[1]
    return pl.pallas_call(
        _mlp_mid_body,
        grid=(grid,),
        in_specs=[pl.BlockSpec((_BLK, cin), lambda i: (i, 0)),
                  pl.BlockSpec((1, cin), lambda i: (0, 0)),
                  pl.BlockSpec((1, cin), lambda i: (0, 0)),
                  pl.BlockSpec((cout, cin), lambda i: (0, 0)),
                  pl.BlockSpec((1, cout), lambda i: (0, 0))],
        out_specs=[pl.BlockSpec((_BLK, cout), lambda i: (i, 0)),
                   pl.BlockSpec((8, cout), lambda i: (0, 0))],
        out_shape=[jax.ShapeDtypeStruct((_ROWS, cout), jnp.float32),
                   jax.ShapeDtypeStruct((8, cout), jnp.float32)],
    )(y, scale.reshape(1, cin), shift.reshape(1, cin), w, b.reshape(1, cout))


def _run_mlp_out(y3, scale, shift):
    grid = _ROWS // 1024
    return pl.pallas_call(
        _mlp_out_body,
        grid=(grid,),
        in_specs=[pl.BlockSpec((1024, C3), lambda i: (i, 0)),
                  pl.BlockSpec((1, C3), lambda i: (0, 0)),
                  pl.BlockSpec((1, C3), lambda i: (0, 0))],
        out_specs=pl.BlockSpec((32, C3), lambda i: (i, 0)),
        out_shape=jax.ShapeDtypeStruct((B_ * NPT, C3), jnp.float32),
    )(y3, scale.reshape(1, C3), shift.reshape(1, C3))


def _bn_fold(s, g, bt):
    rn = jnp.float32(_ROWS)
    mean = s[0] / rn
    var = s[1] / rn - mean * mean
    inv = 1.0 / jnp.sqrt(var + 1e-5)
    scale = g * inv
    shift = bt - mean * scale
    return scale, shift


def kernel(xyz, points, seed_inds, W1, b1, g1, bt1, W2, b2, g2, bt2,
           W3, b3, g3, bt3):
    cent, coord = _run_fps(xyz)
    fpsi = cent[:, :, 0]
    nxyz = coord[:, :, :4].reshape(B_, NPT * 4)

    ptst = jnp.transpose(points, (0, 2, 1))
    xyzt = jnp.transpose(xyz, (0, 2, 1))
    tb = jnp.concatenate(
        [ptst, xyzt, jnp.zeros((B_, N_, RW - D_ - 3), jnp.float32)],
        axis=2).reshape(B_ * N_, RW)
    w1e = jnp.concatenate(
        [W1[:, 3:], W1[:, :3], jnp.zeros((D_, RW - D_ - 3), jnp.float32)], axis=1)
    t1 = _run_t1(tb, w1e, b1)

    gx, seedo = _run_sc(xyz.reshape(-1), nxyz.reshape(-1), fpsi.reshape(-1),
                        seed_inds.reshape(-1), t1)
    seedo = seedo.reshape(B_, NPT)

    nx4 = coord[:, :, :4].reshape(B_ * NPT, 4)
    w1x4 = jnp.pad(W1[:, :3], ((0, 0), (0, 1)))
    y1, s1 = _run_mlp1(gx, nx4, w1x4)
    sc1, sh1 = _bn_fold(s1, g1, bt1)
    y2, s2 = _run_mlp_mid(y1, sc1, sh1, W2, b2, D_)
    sc2, sh2 = _bn_fold(s2, g2, bt2)
    y3, s3 = _run_mlp_mid(y2, sc2, sh2, W3, b3, C3)
    sc3, sh3 = _bn_fold(s3, g3, bt3)
    xo = _run_mlp_out(y3, sc3, sh3)

    new_xyz = jnp.transpose(coord[:, :, :3], (0, 2, 1))
    x_out = jnp.transpose(xo.reshape(B_, NPT, C3), (0, 2, 1))
    return (new_xyz, x_out, seedo)


# speedup vs baseline: 11.3771x; 1.2737x over previous
import functools

import numpy as np
import jax
import jax.numpy as jnp
from jax import lax
from jax.experimental import pallas as pl
from jax.experimental.pallas import tpu as pltpu
from jax.experimental.pallas import tpu_sc as plsc

B_ = 4
N_ = 16384
D_ = 128
NPT = 1024
NS = 32
RAD2 = np.float32(0.4 ** 2)
NW = 32
NQT = NPT // NW
RW = 144
C3 = 256


def _fps_body(xyz_ref, cent_ref, coord_ref, dist_ref):
    x = xyz_ref[:, 0]
    y = xyz_ref[:, 1]
    z = xyz_ref[:, 2]
    pos = (lax.broadcasted_iota(jnp.int32, (B_, 128, 128), 1) * 128
           + lax.broadcasted_iota(jnp.int32, (B_, 128, 128), 2))
    lane = lax.broadcasted_iota(jnp.int32, (B_, 1, 128), 2)
    dist_ref[...] = jnp.full((B_, 128, 128), 1e10, jnp.float32)

    def body(i, far):
        eq = (pos == far).astype(jnp.float32)
        cx = jnp.sum(jnp.sum(x * eq, axis=2, keepdims=True), axis=1, keepdims=True)
        cy = jnp.sum(jnp.sum(y * eq, axis=2, keepdims=True), axis=1, keepdims=True)
        cz = jnp.sum(jnp.sum(z * eq, axis=2, keepdims=True), axis=1, keepdims=True)
        cent_ref[:, pl.ds(i, 1), :] = jnp.broadcast_to(far, (B_, 1, 128))
        crow = jnp.where(lane == 0, cx,
                         jnp.where(lane == 1, cy,
                                   jnp.where(lane == 2, cz, 0.0)))
        coord_ref[:, pl.ds(i, 1), :] = crow
        dx = x - cx
        dy = y - cy
        dz = z - cz
        d = (dx * dx + dy * dy) + dz * dz
        dist = jnp.minimum(dist_ref[...], d)
        dist_ref[...] = dist
        m = jnp.max(jnp.max(dist, axis=2, keepdims=True), axis=1, keepdims=True)
        cand = jnp.where(dist == m, pos, jnp.int32(2 ** 30))
        return jnp.min(jnp.min(cand, axis=2, keepdims=True), axis=1, keepdims=True)

    lax.fori_loop(0, NPT, body, jnp.zeros((B_, 1, 1), jnp.int32))


def _run_fps(xyz):
    xyz4 = xyz.reshape(B_, 3, 128, 128)
    return pl.pallas_call(
        _fps_body,
        out_shape=[jax.ShapeDtypeStruct((B_, NPT, 128), jnp.int32),
                   jax.ShapeDtypeStruct((B_, NPT, 128), jnp.float32)],
        scratch_shapes=[pltpu.VMEM((B_, 128, 128), jnp.float32)],
    )(xyz4)


def _sc_body(xyz_hbm, nxyz_hbm, fpsi_hbm, seeds_hbm, ptab_hbm,
             gx_hbm, seedo_hbm,
             xtab, ytab, ztab, ctab, qbuf, fpsb, seedtab, seedout,
             idx_all, cntbuf, gidx, rows, gsem):
    cid = lax.axis_index("c")
    sid = lax.axis_index("s")
    wid = sid * 2 + cid
    q0 = wid * NQT
    iota16 = lax.iota(jnp.int32, 16)

    for b in range(B_):
        pltpu.sync_copy(xyz_hbm.at[pl.ds((b * 3 + 0) * N_, N_)], xtab)
        pltpu.sync_copy(xyz_hbm.at[pl.ds((b * 3 + 1) * N_, N_)], ytab)
        pltpu.sync_copy(xyz_hbm.at[pl.ds((b * 3 + 2) * N_, N_)], ztab)
        pltpu.sync_copy(seeds_hbm.at[pl.ds(b * N_, N_)], seedtab)
        pltpu.sync_copy(nxyz_hbm.at[pl.ds((b * NPT + q0) * 4, NQT * 4)], qbuf)
        pltpu.sync_copy(fpsi_hbm.at[pl.ds(b * NPT + q0, NQT)], fpsb)

        def ct_body(i, _):
            xv = xtab[pl.ds(i * 16, 16)]
            yv = ytab[pl.ds(i * 16, 16)]
            zv = ztab[pl.ds(i * 16, 16)]
            ctab[pl.ds(i * 16, 16)] = (xv * xv + yv * yv) + zv * zv
            return 0

        lax.fori_loop(0, N_ // 16, ct_body, 0)

        for c in range(NQT // 16):
            iv = fpsb[pl.ds(c * 16, 16)]
            seedout[pl.ds(c * 16, 16)] = plsc.load_gather(seedtab, [iv])
        pltpu.sync_copy(seedout, seedo_hbm.at[pl.ds(b * NPT + q0, NQT)])

        for g in range(NQT // 16):
            qxv = plsc.load_gather(qbuf, [iota16 * 4 + (g * 64 + 0)])
            qyv = plsc.load_gather(qbuf, [iota16 * 4 + (g * 64 + 1)])
            qzv = plsc.load_gather(qbuf, [iota16 * 4 + (g * 64 + 2)])
            sqv = (qxv * qxv + qyv * qyv) + qzv * qzv
            qbase = (iota16 + g * 16) * NS

            def cond(st):
                j, cnt = st
                return jnp.logical_and(jnp.any(cnt < NS), j < N_)

            def sbody(st):
                j, cnt = st
                for u in range(16):
                    jsp = jnp.zeros((16,), jnp.int32) + (j + u)
                    xj = plsc.load_gather(xtab, [jsp])
                    yj = plsc.load_gather(ytab, [jsp])
                    zj = plsc.load_gather(ztab, [jsp])
                    cj = plsc.load_gather(ctab, [jsp])
                    m = (qxv * xj + qyv * yj) + qzv * zj
                    d = (m * jnp.float32(-2.0) + sqv) + cj
                    hit = jnp.logical_and(d <= RAD2, cnt < NS)
                    plsc.store_scatter(idx_all, [qbase + cnt], jsp, mask=hit)
                    cnt = cnt + jnp.where(hit, 1, 0).astype(jnp.int32)
                return (j + 16, cnt)

            _, cntf = lax.while_loop(cond, sbody, (jnp.int32(0), jnp.zeros((16,), jnp.int32)))
            cntbuf[pl.ds(g * 16, 16)] = cntf

        def pad_body(q, _):
            cq = plsc.load_gather(cntbuf, [jnp.zeros((16,), jnp.int32) + q])
            first = plsc.load_gather(idx_all, [jnp.zeros((16,), jnp.int32) + q * NS])
            for c in range(NS // 16):
                sl = idx_all[pl.ds(q * NS + c * 16, 16)]
                slot = iota16 + (c * 16)
                idxv = jnp.where(slot < cq, sl, first)
                gidx[pl.ds(q * NS + c * 16, 16)] = idxv + b * N_
            return 0

        lax.fori_loop(0, NQT, pad_body, 0)

        row0 = (b * NPT + q0) * NS

        def g_body(t, _):
            pltpu.async_copy(ptab_hbm.at[gidx.at[pl.ds(t * 64, 64)]], rows, gsem).wait()
            pltpu.sync_copy(rows, gx_hbm.at[pl.ds(row0 + t * 64, 64)])
            return 0

        lax.fori_loop(0, NQT * NS // 64, g_body, 0)


def _run_sc(xyz, nxyz, fpsi, seed_inds, ptab):
    mesh = plsc.VectorSubcoreMesh(core_axis_name="c", subcore_axis_name="s")
    call = functools.partial(
        pl.kernel,
        mesh=mesh,
        compiler_params=pltpu.CompilerParams(needs_layout_passes=False),
        out_type=[jax.ShapeDtypeStruct((B_ * NPT * NS, D_), jnp.float32),
                  jax.ShapeDtypeStruct((B_ * NPT,), jnp.int32)],
        scratch_types=[
            pltpu.VMEM((N_,), jnp.float32),
            pltpu.VMEM((N_,), jnp.float32),
            pltpu.VMEM((N_,), jnp.float32),
            pltpu.VMEM((N_,), jnp.float32),
            pltpu.VMEM((NQT * 4,), jnp.float32),
            pltpu.VMEM((NQT,), jnp.int32),
            pltpu.VMEM((N_,), jnp.int32),
            pltpu.VMEM((NQT,), jnp.int32),
            pltpu.VMEM((NQT * NS + 24,), jnp.int32),
            pltpu.VMEM((NQT,), jnp.int32),
            pltpu.VMEM((NQT * NS,), jnp.int32),
            pltpu.VMEM((64, D_), jnp.float32),
            pltpu.SemaphoreType.DMA,
        ],
    )(_sc_body)
    return call(xyz, nxyz, fpsi, seed_inds, ptab)


_ROWS = B_ * NPT * NS
_BLK = 512


def _mm(a, b):
    return lax.dot_general(a, b, (((1,), (1,)), ((), ())),
                           precision=lax.Precision.HIGHEST,
                           preferred_element_type=jnp.float32)


def _stats_accum(s_ref, y):
    @pl.when(pl.program_id(0) == 0)
    def _():
        s_ref[...] = jnp.zeros_like(s_ref)

    s_ref[0:1, :] += jnp.sum(y, axis=0, keepdims=True)
    s_ref[1:2, :] += jnp.sum(y * y, axis=0, keepdims=True)


def _t1_body(x_ref, w_ref, b_ref, y_ref):
    y_ref[...] = _mm(x_ref[...], w_ref[...]) + b_ref[...]


def _mlp1_body(gx_ref, nx_ref, wx_ref, y_ref, s_ref):
    corr = _mm(nx_ref[...], wx_ref[...])
    corrb = jnp.broadcast_to(corr.reshape(16, 1, D_), (16, NS, D_)).reshape(_BLK, D_)
    y = gx_ref[...] - corrb
    y_ref[...] = y
    _stats_accum(s_ref, y)


def _mlp_mid_body(x_ref, sc_ref, sh_ref, w_ref, b_ref, y_ref, s_ref):
    h = jnp.maximum(x_ref[...] * sc_ref[...] + sh_ref[...], 0.0)
    y = _mm(h, w_ref[...]) + b_ref[...]
    y_ref[...] = y
    _stats_accum(s_ref, y)


def _mlp_out_body(x_ref, sc_ref, sh_ref, o_ref):
    h = jnp.maximum(x_ref[...] * sc_ref[...] + sh_ref[...], 0.0)
    o_ref[...] = jnp.max(h.reshape(32, NS, C3), axis=1)


def _run_t1(tb, w1e, b1):
    grid = B_ * N_ // _BLK
    return pl.pallas_call(
        _t1_body,
        grid=(grid,),
        in_specs=[pl.BlockSpec((_BLK, RW), lambda i: (i, 0)),
                  pl.BlockSpec((D_, RW), lambda i: (0, 0)),
                  pl.BlockSpec((1, D_), lambda i: (0, 0))],
        out_specs=pl.BlockSpec((_BLK, D_), lambda i: (i, 0)),
        out_shape=jax.ShapeDtypeStruct((B_ * N_, D_), jnp.float32),
    )(tb, w1e, b1.reshape(1, D_))


def _run_mlp1(gx, nx4, w1x4):
    grid = _ROWS // _BLK
    return pl.pallas_call(
        _mlp1_body,
        grid=(grid,),
        in_specs=[pl.BlockSpec((_BLK, D_), lambda i: (i, 0)),
                  pl.BlockSpec((16, 4), lambda i: (i, 0)),
                  pl.BlockSpec((D_, 4), lambda i: (0, 0))],
        out_specs=[pl.BlockSpec((_BLK, D_), lambda i: (i, 0)),
                   pl.BlockSpec((8, D_), lambda i: (0, 0))],
        out_shape=[jax.ShapeDtypeStruct((_ROWS, D_), jnp.float32),
                   jax.ShapeDtypeStruct((8, D_), jnp.float32)],
    )(gx, nx4, w1x4)


def _run_mlp_mid(y, scale, shift, w, b, cout):
    grid = _ROWS // _BLK
    cin = y.shape[1]
    return pl.pallas_call(
        _mlp_mid_body,
        grid=(grid,),
        in_specs=[pl.BlockSpec((_BLK, cin), lambda i: (i, 0)),
                  pl.BlockSpec((1, cin), lambda i: (0, 0)),
                  pl.BlockSpec((1, cin), lambda i: (0, 0)),
                  pl.BlockSpec((cout, cin), lambda i: (0, 0)),
                  pl.BlockSpec((1, cout), lambda i: (0, 0))],
        out_specs=[pl.BlockSpec((_BLK, cout), lambda i: (i, 0)),
                   pl.BlockSpec((8, cout), lambda i: (0, 0))],
        out_shape=[jax.ShapeDtypeStruct((_ROWS, cout), jnp.float32),
                   jax.ShapeDtypeStruct((8, cout), jnp.float32)],
    )(y, scale.reshape(1, cin), shift.reshape(1, cin), w, b.reshape(1, cout))


def _run_mlp_out(y3, scale, shift):
    grid = _ROWS // 1024
    return pl.pallas_call(
        _mlp_out_body,
        grid=(grid,),
        in_specs=[pl.BlockSpec((1024, C3), lambda i: (i, 0)),
                  pl.BlockSpec((1, C3), lambda i: (0, 0)),
                  pl.BlockSpec((1, C3), lambda i: (0, 0))],
        out_specs=pl.BlockSpec((32, C3), lambda i: (i, 0)),
        out_shape=jax.ShapeDtypeStruct((B_ * NPT, C3), jnp.float32),
    )(y3, scale.reshape(1, C3), shift.reshape(1, C3))


def _bn_fold(s, g, bt):
    rn = jnp.float32(_ROWS)
    mean = s[0] / rn
    var = s[1] / rn - mean * mean
    inv = 1.0 / jnp.sqrt(var + 1e-5)
    scale = g * inv
    shift = bt - mean * scale
    return scale, shift


def kernel(xyz, points, seed_inds, W1, b1, g1, bt1, W2, b2, g2, bt2,
           W3, b3, g3, bt3):
    cent, coord = _run_fps(xyz)
    fpsi = cent[:, :, 0]
    nxyz = coord[:, :, :4].reshape(B_, NPT * 4)

    ptst = jnp.transpose(points, (0, 2, 1))
    xyzt = jnp.transpose(xyz, (0, 2, 1))
    tb = jnp.concatenate(
        [ptst, xyzt, jnp.zeros((B_, N_, RW - D_ - 3), jnp.float32)],
        axis=2).reshape(B_ * N_, RW)
    w1e = jnp.concatenate(
        [W1[:, 3:], W1[:, :3], jnp.zeros((D_, RW - D_ - 3), jnp.float32)], axis=1)
    t1 = _run_t1(tb, w1e, b1)

    gx, seedo = _run_sc(xyz.reshape(-1), nxyz.reshape(-1), fpsi.reshape(-1),
                        seed_inds.reshape(-1), t1)
    seedo = seedo.reshape(B_, NPT)

    nx4 = coord[:, :, :4].reshape(B_ * NPT, 4)
    w1x4 = jnp.pad(W1[:, :3], ((0, 0), (0, 1)))
    y1, s1 = _run_mlp1(gx, nx4, w1x4)
    sc1, sh1 = _bn_fold(s1, g1, bt1)
    y2, s2 = _run_mlp_mid(y1, sc1, sh1, W2, b2, D_)
    sc2, sh2 = _bn_fold(s2, g2, bt2)
    y3, s3 = _run_mlp_mid(y2, sc2, sh2, W3, b3, C3)
    sc3, sh3 = _bn_fold(s3, g3, bt3)
    xo = _run_mlp_out(y3, sc3, sh3)

    new_xyz = jnp.transpose(coord[:, :, :3], (0, 2, 1))
    x_out = jnp.transpose(xo.reshape(B_, NPT, C3), (0, 2, 1))
    return (new_xyz, x_out, seedo)


# speedup vs baseline: 17.0777x; 1.5011x over previous
import functools

import numpy as np
import jax
import jax.numpy as jnp
from jax import lax
from jax.experimental import pallas as pl
from jax.experimental.pallas import tpu as pltpu
from jax.experimental.pallas import tpu_sc as plsc

B_ = 4
N_ = 16384
D_ = 128
NPT = 1024
NS = 32
RAD2 = np.float32(0.4 ** 2)
NW = 32
NQT = NPT // NW
RW = 144
C3 = 256


def _fps_body(xyz_ref, cent_ref, coord_ref, dist_ref):
    x = xyz_ref[:, 0]
    y = xyz_ref[:, 1]
    z = xyz_ref[:, 2]
    pos = (lax.broadcasted_iota(jnp.int32, (B_, 128, 128), 1) * 128
           + lax.broadcasted_iota(jnp.int32, (B_, 128, 128), 2))
    lane = lax.broadcasted_iota(jnp.int32, (B_, 1, 128), 2)
    dist_ref[...] = jnp.full((B_, 128, 128), 1e10, jnp.float32)

    def body(i, far):
        eq = (pos == far).astype(jnp.float32)
        cx = jnp.sum(jnp.sum(x * eq, axis=2, keepdims=True), axis=1, keepdims=True)
        cy = jnp.sum(jnp.sum(y * eq, axis=2, keepdims=True), axis=1, keepdims=True)
        cz = jnp.sum(jnp.sum(z * eq, axis=2, keepdims=True), axis=1, keepdims=True)
        cent_ref[:, pl.ds(i, 1), :] = jnp.broadcast_to(far, (B_, 1, 128))
        crow = jnp.where(lane == 0, cx,
                         jnp.where(lane == 1, cy,
                                   jnp.where(lane == 2, cz, 0.0)))
        coord_ref[:, pl.ds(i, 1), :] = crow
        dx = x - cx
        dy = y - cy
        dz = z - cz
        d = (dx * dx + dy * dy) + dz * dz
        dist = jnp.minimum(dist_ref[...], d)
        dist_ref[...] = dist
        m = jnp.max(jnp.max(dist, axis=2, keepdims=True), axis=1, keepdims=True)
        cand = jnp.where(dist == m, pos, jnp.int32(2 ** 30))
        return jnp.min(jnp.min(cand, axis=2, keepdims=True), axis=1, keepdims=True)

    lax.fori_loop(0, NPT, body, jnp.zeros((B_, 1, 1), jnp.int32))


def _run_fps(xyz):
    xyz4 = xyz.reshape(B_, 3, 128, 128)
    return pl.pallas_call(
        _fps_body,
        out_shape=[jax.ShapeDtypeStruct((B_, NPT, 128), jnp.int32),
                   jax.ShapeDtypeStruct((B_, NPT, 128), jnp.float32)],
        scratch_shapes=[pltpu.VMEM((B_, 128, 128), jnp.float32)],
    )(xyz4)


def _sc_body(xyz_hbm, nxyz_hbm, fpsi_hbm, seeds_hbm, ptab_hbm,
             gx_hbm, seedo_hbm,
             xtab, ytab, ztab, ctab, qbuf, fpsb, seedtab, seedout,
             idx_all, cntbuf, gidx, rows, gsem):
    cid = lax.axis_index("c")
    sid = lax.axis_index("s")
    wid = sid * 2 + cid
    q0 = wid * NQT
    iota16 = lax.iota(jnp.int32, 16)

    for b in range(B_):
        pltpu.sync_copy(xyz_hbm.at[pl.ds((b * 3 + 0) * N_, N_)], xtab)
        pltpu.sync_copy(xyz_hbm.at[pl.ds((b * 3 + 1) * N_, N_)], ytab)
        pltpu.sync_copy(xyz_hbm.at[pl.ds((b * 3 + 2) * N_, N_)], ztab)
        pltpu.sync_copy(seeds_hbm.at[pl.ds(b * N_, N_)], seedtab)
        pltpu.sync_copy(nxyz_hbm.at[pl.ds((b * NPT + q0) * 4, NQT * 4)], qbuf)
        pltpu.sync_copy(fpsi_hbm.at[pl.ds(b * NPT + q0, NQT)], fpsb)

        def ct_body(i, _):
            xv = xtab[pl.ds(i * 16, 16)]
            yv = ytab[pl.ds(i * 16, 16)]
            zv = ztab[pl.ds(i * 16, 16)]
            ctab[pl.ds(i * 16, 16)] = (xv * xv + yv * yv) + zv * zv
            return 0

        lax.fori_loop(0, N_ // 16, ct_body, 0)

        for c in range(NQT // 16):
            iv = fpsb[pl.ds(c * 16, 16)]
            seedout[pl.ds(c * 16, 16)] = plsc.load_gather(seedtab, [iv])
        pltpu.sync_copy(seedout, seedo_hbm.at[pl.ds(b * NPT + q0, NQT)])

        qx0 = plsc.load_gather(qbuf, [iota16 * 4 + 0])
        qy0 = plsc.load_gather(qbuf, [iota16 * 4 + 1])
        qz0 = plsc.load_gather(qbuf, [iota16 * 4 + 2])
        sq0 = (qx0 * qx0 + qy0 * qy0) + qz0 * qz0
        qx1 = plsc.load_gather(qbuf, [iota16 * 4 + 64])
        qy1 = plsc.load_gather(qbuf, [iota16 * 4 + 65])
        qz1 = plsc.load_gather(qbuf, [iota16 * 4 + 66])
        sq1 = (qx1 * qx1 + qy1 * qy1) + qz1 * qz1
        qb0 = iota16 * NS
        qb1 = (iota16 + 16) * NS
        _dn = lax.GatherDimensionNumbers(
            offset_dims=(), collapsed_slice_dims=(0,), start_index_map=(0,))

        def _bcast(v, usp):
            return lax.gather(v, usp[:, None], _dn, (1,),
                              mode=lax.GatherScatterMode.PROMISE_IN_BOUNDS)

        def cond(st):
            j, cnt0, cnt1 = st
            more = jnp.logical_or(jnp.any(cnt0 < NS), jnp.any(cnt1 < NS))
            return jnp.logical_and(more, j < N_)

        def sbody(st):
            j, cnt0, cnt1 = st
            xv = xtab[pl.ds(j, 16)]
            yv = ytab[pl.ds(j, 16)]
            zv = ztab[pl.ds(j, 16)]
            cv = ctab[pl.ds(j, 16)]
            for u in range(16):
                usp = jnp.full((16,), u, jnp.int32)
                xj = _bcast(xv, usp)
                yj = _bcast(yv, usp)
                zj = _bcast(zv, usp)
                cj = _bcast(cv, usp)
                jsp = jnp.zeros((16,), jnp.int32) + (j + u)
                m0 = (qx0 * xj + qy0 * yj) + qz0 * zj
                d0 = (m0 * jnp.float32(-2.0) + sq0) + cj
                hit0 = jnp.logical_and(d0 <= RAD2, cnt0 < NS)
                plsc.store_scatter(idx_all, [qb0 + cnt0], jsp, mask=hit0)
                cnt0 = cnt0 + jnp.where(hit0, 1, 0).astype(jnp.int32)
                m1 = (qx1 * xj + qy1 * yj) + qz1 * zj
                d1 = (m1 * jnp.float32(-2.0) + sq1) + cj
                hit1 = jnp.logical_and(d1 <= RAD2, cnt1 < NS)
                plsc.store_scatter(idx_all, [qb1 + cnt1], jsp, mask=hit1)
                cnt1 = cnt1 + jnp.where(hit1, 1, 0).astype(jnp.int32)
            return (j + 16, cnt0, cnt1)

        z16 = jnp.zeros((16,), jnp.int32)
        _, cf0, cf1 = lax.while_loop(cond, sbody, (jnp.int32(0), z16, z16))
        cntbuf[pl.ds(0, 16)] = cf0
        cntbuf[pl.ds(16, 16)] = cf1

        def pad_body(q, _):
            cq = plsc.load_gather(cntbuf, [jnp.zeros((16,), jnp.int32) + q])
            first = plsc.load_gather(idx_all, [jnp.zeros((16,), jnp.int32) + q * NS])
            for c in range(NS // 16):
                sl = idx_all[pl.ds(q * NS + c * 16, 16)]
                slot = iota16 + (c * 16)
                idxv = jnp.where(slot < cq, sl, first)
                gidx[pl.ds(q * NS + c * 16, 16)] = idxv + b * N_
            return 0

        lax.fori_loop(0, NQT, pad_body, 0)

        row0 = (b * NPT + q0) * NS

        def g_body(t, _):
            pltpu.async_copy(ptab_hbm.at[gidx.at[pl.ds(t * 64, 64)]], rows, gsem).wait()
            pltpu.sync_copy(rows, gx_hbm.at[pl.ds(row0 + t * 64, 64)])
            return 0

        lax.fori_loop(0, NQT * NS // 64, g_body, 0)


def _run_sc(xyz, nxyz, fpsi, seed_inds, ptab):
    mesh = plsc.VectorSubcoreMesh(core_axis_name="c", subcore_axis_name="s")
    call = functools.partial(
        pl.kernel,
        mesh=mesh,
        compiler_params=pltpu.CompilerParams(needs_layout_passes=False),
        out_type=[jax.ShapeDtypeStruct((B_ * NPT * NS, D_), jnp.float32),
                  jax.ShapeDtypeStruct((B_ * NPT,), jnp.int32)],
        scratch_types=[
            pltpu.VMEM((N_,), jnp.float32),
            pltpu.VMEM((N_,), jnp.float32),
            pltpu.VMEM((N_,), jnp.float32),
            pltpu.VMEM((N_,), jnp.float32),
            pltpu.VMEM((NQT * 4,), jnp.float32),
            pltpu.VMEM((NQT,), jnp.int32),
            pltpu.VMEM((N_,), jnp.int32),
            pltpu.VMEM((NQT,), jnp.int32),
            pltpu.VMEM((NQT * NS + 24,), jnp.int32),
            pltpu.VMEM((NQT,), jnp.int32),
            pltpu.VMEM((NQT * NS,), jnp.int32),
            pltpu.VMEM((64, D_), jnp.float32),
            pltpu.SemaphoreType.DMA,
        ],
    )(_sc_body)
    return call(xyz, nxyz, fpsi, seed_inds, ptab)


_ROWS = B_ * NPT * NS
_BLK = 512


def _mm(a, b):
    return lax.dot_general(a, b, (((1,), (1,)), ((), ())),
                           precision=lax.Precision.HIGHEST,
                           preferred_element_type=jnp.float32)


def _stats_accum(s_ref, y):
    @pl.when(pl.program_id(0) == 0)
    def _():
        s_ref[...] = jnp.zeros_like(s_ref)

    s_ref[0:1, :] += jnp.sum(y, axis=0, keepdims=True)
    s_ref[1:2, :] += jnp.sum(y * y, axis=0, keepdims=True)


def _t1_body(x_ref, w_ref, b_ref, y_ref):
    y_ref[...] = _mm(x_ref[...], w_ref[...]) + b_ref[...]


def _mlp1_body(gx_ref, nx_ref, wx_ref, y_ref, s_ref):
    corr = _mm(nx_ref[...], wx_ref[...])
    corrb = jnp.broadcast_to(corr.reshape(16, 1, D_), (16, NS, D_)).reshape(_BLK, D_)
    y = gx_ref[...] - corrb
    y_ref[...] = y
    _stats_accum(s_ref, y)


def _mlp_mid_body(x_ref, sc_ref, sh_ref, w_ref, b_ref, y_ref, s_ref):
    h = jnp.maximum(x_ref[...] * sc_ref[...] + sh_ref[...], 0.0)
    y = _mm(h, w_ref[...]) + b_ref[...]
    y_ref[...] = y
    _stats_accum(s_ref, y)


def _mlp_out_body(x_ref, sc_ref, sh_ref, o_ref):
    h = jnp.maximum(x_ref[...] * sc_ref[...] + sh_ref[...], 0.0)
    o_ref[...] = jnp.max(h.reshape(32, NS, C3), axis=1)


def _run_t1(tb, w1e, b1):
    grid = B_ * N_ // _BLK
    return pl.pallas_call(
        _t1_body,
        grid=(grid,),
        in_specs=[pl.BlockSpec((_BLK, RW), lambda i: (i, 0)),
                  pl.BlockSpec((D_, RW), lambda i: (0, 0)),
                  pl.BlockSpec((1, D_), lambda i: (0, 0))],
        out_specs=pl.BlockSpec((_BLK, D_), lambda i: (i, 0)),
        out_shape=jax.ShapeDtypeStruct((B_ * N_, D_), jnp.float32),
    )(tb, w1e, b1.reshape(1, D_))


def _run_mlp1(gx, nx4, w1x4):
    grid = _ROWS // _BLK
    return pl.pallas_call(
        _mlp1_body,
        grid=(grid,),
        in_specs=[pl.BlockSpec((_BLK, D_), lambda i: (i, 0)),
                  pl.BlockSpec((16, 4), lambda i: (i, 0)),
                  pl.BlockSpec((D_, 4), lambda i: (0, 0))],
        out_specs=[pl.BlockSpec((_BLK, D_), lambda i: (i, 0)),
                   pl.BlockSpec((8, D_), lambda i: (0, 0))],
        out_shape=[jax.ShapeDtypeStruct((_ROWS, D_), jnp.float32),
                   jax.ShapeDtypeStruct((8, D_), jnp.float32)],
    )(gx, nx4, w1x4)


def _run_mlp_mid(y, scale, shift, w, b, cout):
    grid = _ROWS // _BLK
    cin = y.shape[1]
    return pl.pallas_call(
        _mlp_mid_body,
        grid=(grid,),
        in_specs=[pl.BlockSpec((_BLK, cin), lambda i: (i, 0)),
                  pl.BlockSpec((1, cin), lambda i: (0, 0)),
                  pl.BlockSpec((1, cin), lambda i: (0, 0)),
                  pl.BlockSpec((cout, cin), lambda i: (0, 0)),
                  pl.BlockSpec((1, cout), lambda i: (0, 0))],
        out_specs=[pl.BlockSpec((_BLK, cout), lambda i: (i, 0)),
                   pl.BlockSpec((8, cout), lambda i: (0, 0))],
        out_shape=[jax.ShapeDtypeStruct((_ROWS, cout), jnp.float32),
                   jax.ShapeDtypeStruct((8, cout), jnp.float32)],
    )(y, scale.reshape(1, cin), shift.reshape(1, cin), w, b.reshape(1, cout))


def _run_mlp_out(y3, scale, shift):
    grid = _ROWS // 1024
    return pl.pallas_call(
        _mlp_out_body,
        grid=(grid,),
        in_specs=[pl.BlockSpec((1024, C3), lambda i: (i, 0)),
                  pl.BlockSpec((1, C3), lambda i: (0, 0)),
                  pl.BlockSpec((1, C3), lambda i: (0, 0))],
        out_specs=pl.BlockSpec((32, C3), lambda i: (i, 0)),
        out_shape=jax.ShapeDtypeStruct((B_ * NPT, C3), jnp.float32),
    )(y3, scale.reshape(1, C3), shift.reshape(1, C3))


def _bn_fold(s, g, bt):
    rn = jnp.float32(_ROWS)
    mean = s[0] / rn
    var = s[1] / rn - mean * mean
    inv = 1.0 / jnp.sqrt(var + 1e-5)
    scale = g * inv
    shift = bt - mean * scale
    return scale, shift


def kernel(xyz, points, seed_inds, W1, b1, g1, bt1, W2, b2, g2, bt2,
           W3, b3, g3, bt3):
    cent, coord = _run_fps(xyz)
    fpsi = cent[:, :, 0]
    nxyz = coord[:, :, :4].reshape(B_, NPT * 4)

    ptst = jnp.transpose(points, (0, 2, 1))
    xyzt = jnp.transpose(xyz, (0, 2, 1))
    tb = jnp.concatenate(
        [ptst, xyzt, jnp.zeros((B_, N_, RW - D_ - 3), jnp.float32)],
        axis=2).reshape(B_ * N_, RW)
    w1e = jnp.concatenate(
        [W1[:, 3:], W1[:, :3], jnp.zeros((D_, RW - D_ - 3), jnp.float32)], axis=1)
    t1 = _run_t1(tb, w1e, b1)

    gx, seedo = _run_sc(xyz.reshape(-1), nxyz.reshape(-1), fpsi.reshape(-1),
                        seed_inds.reshape(-1), t1)
    seedo = seedo.reshape(B_, NPT)

    nx4 = coord[:, :, :4].reshape(B_ * NPT, 4)
    w1x4 = jnp.pad(W1[:, :3], ((0, 0), (0, 1)))
    y1, s1 = _run_mlp1(gx, nx4, w1x4)
    sc1, sh1 = _bn_fold(s1, g1, bt1)
    y2, s2 = _run_mlp_mid(y1, sc1, sh1, W2, b2, D_)
    sc2, sh2 = _bn_fold(s2, g2, bt2)
    y3, s3 = _run_mlp_mid(y2, sc2, sh2, W3, b3, C3)
    sc3, sh3 = _bn_fold(s3, g3, bt3)
    xo = _run_mlp_out(y3, sc3, sh3)

    new_xyz = jnp.transpose(coord[:, :, :3], (0, 2, 1))
    x_out = jnp.transpose(xo.reshape(B_, NPT, C3), (0, 2, 1))
    return (new_xyz, x_out, seedo)


# speedup vs baseline: 17.8632x; 1.0460x over previous
import functools

import numpy as np
import jax
import jax.numpy as jnp
from jax import lax
from jax.experimental import pallas as pl
from jax.experimental.pallas import tpu as pltpu
from jax.experimental.pallas import tpu_sc as plsc

B_ = 4
N_ = 16384
D_ = 128
NPT = 1024
NS = 32
RAD2 = np.float32(0.4 ** 2)
NW = 32
NQT = NPT // NW
RW = 144
C3 = 256


def _fps_body(xyz_ref, cent_ref, coord_ref, dist_ref):
    x = xyz_ref[:, 0]
    y = xyz_ref[:, 1]
    z = xyz_ref[:, 2]
    pos = (lax.broadcasted_iota(jnp.int32, (B_, 128, 128), 1) * 128
           + lax.broadcasted_iota(jnp.int32, (B_, 128, 128), 2))
    lane = lax.broadcasted_iota(jnp.int32, (B_, 1, 128), 2)
    dist_ref[...] = jnp.full((B_, 128, 128), 1e10, jnp.float32)

    def _sum2(v):
        return jnp.sum(jnp.sum(v, axis=2, keepdims=True), axis=1, keepdims=True)

    eq0 = (pos == 0).astype(jnp.float32)
    cx0 = _sum2(x * eq0)
    cy0 = _sum2(y * eq0)
    cz0 = _sum2(z * eq0)

    def body(i, st):
        far, cx, cy, cz = st
        cent_ref[:, pl.ds(i, 1), :] = jnp.broadcast_to(far, (B_, 1, 128))
        crow = jnp.where(lane == 0, cx,
                         jnp.where(lane == 1, cy,
                                   jnp.where(lane == 2, cz, 0.0)))
        coord_ref[:, pl.ds(i, 1), :] = crow
        dx = x - cx
        dy = y - cy
        dz = z - cz
        d = (dx * dx + dy * dy) + dz * dz
        dist = jnp.minimum(dist_ref[...], d)
        dist_ref[...] = dist
        m = jnp.max(jnp.max(dist, axis=2, keepdims=True), axis=1, keepdims=True)
        cand = jnp.where(dist == m, pos, jnp.int32(2 ** 30))
        far2 = jnp.min(jnp.min(cand, axis=2, keepdims=True), axis=1, keepdims=True)
        eqf = (cand == far2).astype(jnp.float32)
        return (far2, _sum2(x * eqf), _sum2(y * eqf), _sum2(z * eqf))

    far0 = jnp.zeros((B_, 1, 1), jnp.int32)
    lax.fori_loop(0, NPT, body, (far0, cx0, cy0, cz0))


def _run_fps(xyz):
    xyz4 = xyz.reshape(B_, 3, 128, 128)
    return pl.pallas_call(
        _fps_body,
        out_shape=[jax.ShapeDtypeStruct((B_, NPT, 128), jnp.int32),
                   jax.ShapeDtypeStruct((B_, NPT, 128), jnp.float32)],
        scratch_shapes=[pltpu.VMEM((B_, 128, 128), jnp.float32)],
    )(xyz4)


def _sc_body(xyz_hbm, nxyz_hbm, fpsi_hbm, seeds_hbm, ptab_hbm,
             gx_hbm, seedo_hbm,
             xtab, ytab, ztab, ctab, qbuf, fpsb, seedtab, seedout,
             idx_all, cntbuf, gidx, rows, gsem):
    cid = lax.axis_index("c")
    sid = lax.axis_index("s")
    wid = sid * 2 + cid
    q0 = wid * NQT
    iota16 = lax.iota(jnp.int32, 16)

    for b in range(B_):
        pltpu.sync_copy(xyz_hbm.at[pl.ds((b * 3 + 0) * N_, N_)], xtab)
        pltpu.sync_copy(xyz_hbm.at[pl.ds((b * 3 + 1) * N_, N_)], ytab)
        pltpu.sync_copy(xyz_hbm.at[pl.ds((b * 3 + 2) * N_, N_)], ztab)
        pltpu.sync_copy(seeds_hbm.at[pl.ds(b * N_, N_)], seedtab)
        pltpu.sync_copy(nxyz_hbm.at[pl.ds((b * NPT + q0) * 4, NQT * 4)], qbuf)
        pltpu.sync_copy(fpsi_hbm.at[pl.ds(b * NPT + q0, NQT)], fpsb)

        def ct_body(i, _):
            xv = xtab[pl.ds(i * 16, 16)]
            yv = ytab[pl.ds(i * 16, 16)]
            zv = ztab[pl.ds(i * 16, 16)]
            ctab[pl.ds(i * 16, 16)] = (xv * xv + yv * yv) + zv * zv
            return 0

        lax.fori_loop(0, N_ // 16, ct_body, 0)

        for c in range(NQT // 16):
            iv = fpsb[pl.ds(c * 16, 16)]
            seedout[pl.ds(c * 16, 16)] = plsc.load_gather(seedtab, [iv])
        pltpu.sync_copy(seedout, seedo_hbm.at[pl.ds(b * NPT + q0, NQT)])

        qx0 = plsc.load_gather(qbuf, [iota16 * 4 + 0])
        qy0 = plsc.load_gather(qbuf, [iota16 * 4 + 1])
        qz0 = plsc.load_gather(qbuf, [iota16 * 4 + 2])
        sq0 = (qx0 * qx0 + qy0 * qy0) + qz0 * qz0
        qx1 = plsc.load_gather(qbuf, [iota16 * 4 + 64])
        qy1 = plsc.load_gather(qbuf, [iota16 * 4 + 65])
        qz1 = plsc.load_gather(qbuf, [iota16 * 4 + 66])
        sq1 = (qx1 * qx1 + qy1 * qy1) + qz1 * qz1
        qb0 = iota16 * NS
        qb1 = (iota16 + 16) * NS
        _dn = lax.GatherDimensionNumbers(
            offset_dims=(), collapsed_slice_dims=(0,), start_index_map=(0,))

        def _bcast(v, usp):
            return lax.gather(v, usp[:, None], _dn, (1,),
                              mode=lax.GatherScatterMode.PROMISE_IN_BOUNDS)

        def cond(st):
            j, cnt0, cnt1 = st
            more = jnp.logical_or(jnp.any(cnt0 < NS), jnp.any(cnt1 < NS))
            return jnp.logical_and(more, j < N_)

        def sbody(st):
            j, cnt0, cnt1 = st
            xv = xtab[pl.ds(j, 16)]
            yv = ytab[pl.ds(j, 16)]
            zv = ztab[pl.ds(j, 16)]
            cv = ctab[pl.ds(j, 16)]
            for u in range(16):
                usp = jnp.full((16,), u, jnp.int32)
                xj = _bcast(xv, usp)
                yj = _bcast(yv, usp)
                zj = _bcast(zv, usp)
                cj = _bcast(cv, usp)
                jsp = jnp.zeros((16,), jnp.int32) + (j + u)
                m0 = (qx0 * xj + qy0 * yj) + qz0 * zj
                d0 = (m0 * jnp.float32(-2.0) + sq0) + cj
                hit0 = jnp.logical_and(d0 <= RAD2, cnt0 < NS)
                plsc.store_scatter(idx_all, [qb0 + cnt0], jsp, mask=hit0)
                cnt0 = cnt0 + jnp.where(hit0, 1, 0).astype(jnp.int32)
                m1 = (qx1 * xj + qy1 * yj) + qz1 * zj
                d1 = (m1 * jnp.float32(-2.0) + sq1) + cj
                hit1 = jnp.logical_and(d1 <= RAD2, cnt1 < NS)
                plsc.store_scatter(idx_all, [qb1 + cnt1], jsp, mask=hit1)
                cnt1 = cnt1 + jnp.where(hit1, 1, 0).astype(jnp.int32)
            return (j + 16, cnt0, cnt1)

        z16 = jnp.zeros((16,), jnp.int32)
        _, cf0, cf1 = lax.while_loop(cond, sbody, (jnp.int32(0), z16, z16))
        cntbuf[pl.ds(0, 16)] = cf0
        cntbuf[pl.ds(16, 16)] = cf1

        def pad_body(q, _):
            cq = plsc.load_gather(cntbuf, [jnp.zeros((16,), jnp.int32) + q])
            first = plsc.load_gather(idx_all, [jnp.zeros((16,), jnp.int32) + q * NS])
            for c in range(NS // 16):
                sl = idx_all[pl.ds(q * NS + c * 16, 16)]
                slot = iota16 + (c * 16)
                idxv = jnp.where(slot < cq, sl, first)
                gidx[pl.ds(q * NS + c * 16, 16)] = idxv + b * N_
            return 0

        lax.fori_loop(0, NQT, pad_body, 0)

        row0 = (b * NPT + q0) * NS

        def g_body(t, _):
            pltpu.async_copy(ptab_hbm.at[gidx.at[pl.ds(t * 64, 64)]], rows, gsem).wait()
            pltpu.sync_copy(rows, gx_hbm.at[pl.ds(row0 + t * 64, 64)])
            return 0

        lax.fori_loop(0, NQT * NS // 64, g_body, 0)


def _run_sc(xyz, nxyz, fpsi, seed_inds, ptab):
    mesh = plsc.VectorSubcoreMesh(core_axis_name="c", subcore_axis_name="s")
    call = functools.partial(
        pl.kernel,
        mesh=mesh,
        compiler_params=pltpu.CompilerParams(needs_layout_passes=False),
        out_type=[jax.ShapeDtypeStruct((B_ * NPT * NS, D_), jnp.float32),
                  jax.ShapeDtypeStruct((B_ * NPT,), jnp.int32)],
        scratch_types=[
            pltpu.VMEM((N_,), jnp.float32),
            pltpu.VMEM((N_,), jnp.float32),
            pltpu.VMEM((N_,), jnp.float32),
            pltpu.VMEM((N_,), jnp.float32),
            pltpu.VMEM((NQT * 4,), jnp.float32),
            pltpu.VMEM((NQT,), jnp.int32),
            pltpu.VMEM((N_,), jnp.int32),
            pltpu.VMEM((NQT,), jnp.int32),
            pltpu.VMEM((NQT * NS + 24,), jnp.int32),
            pltpu.VMEM((NQT,), jnp.int32),
            pltpu.VMEM((NQT * NS,), jnp.int32),
            pltpu.VMEM((64, D_), jnp.float32),
            pltpu.SemaphoreType.DMA,
        ],
    )(_sc_body)
    return call(xyz, nxyz, fpsi, seed_inds, ptab)


_ROWS = B_ * NPT * NS
_BLK = 512


def _mm(a, b):
    return lax.dot_general(a, b, (((1,), (1,)), ((), ())),
                           preferred_element_type=jnp.float32)


def _stats_accum(s_ref, y):
    @pl.when(pl.program_id(0) == 0)
    def _():
        s_ref[...] = jnp.zeros_like(s_ref)

    s_ref[0:1, :] += jnp.sum(y, axis=0, keepdims=True)
    s_ref[1:2, :] += jnp.sum(y * y, axis=0, keepdims=True)


def _t1_body(x_ref, w_ref, b_ref, y_ref):
    y_ref[...] = _mm(x_ref[...], w_ref[...]) + b_ref[...]


def _mlp1_body(gx_ref, nx_ref, wx_ref, y_ref, s_ref):
    corr = _mm(nx_ref[...], wx_ref[...])
    corrb = jnp.broadcast_to(corr.reshape(16, 1, D_), (16, NS, D_)).reshape(_BLK, D_)
    y = gx_ref[...] - corrb
    y_ref[...] = y
    _stats_accum(s_ref, y)


def _mlp_mid_body(x_ref, sc_ref, sh_ref, w_ref, b_ref, y_ref, s_ref):
    h = jnp.maximum(x_ref[...] * sc_ref[...] + sh_ref[...], 0.0)
    y = _mm(h, w_ref[...]) + b_ref[...]
    y_ref[...] = y
    _stats_accum(s_ref, y)


def _mlp_out_body(x_ref, sc_ref, sh_ref, o_ref):
    h = jnp.maximum(x_ref[...] * sc_ref[...] + sh_ref[...], 0.0)
    o_ref[...] = jnp.max(h.reshape(32, NS, C3), axis=1)


def _run_t1(tb, w1e, b1):
    grid = B_ * N_ // _BLK
    return pl.pallas_call(
        _t1_body,
        grid=(grid,),
        in_specs=[pl.BlockSpec((_BLK, RW), lambda i: (i, 0)),
                  pl.BlockSpec((D_, RW), lambda i: (0, 0)),
                  pl.BlockSpec((1, D_), lambda i: (0, 0))],
        out_specs=pl.BlockSpec((_BLK, D_), lambda i: (i, 0)),
        out_shape=jax.ShapeDtypeStruct((B_ * N_, D_), jnp.float32),
    )(tb, w1e, b1.reshape(1, D_))


def _run_mlp1(gx, nx4, w1x4):
    grid = _ROWS // _BLK
    return pl.pallas_call(
        _mlp1_body,
        grid=(grid,),
        in_specs=[pl.BlockSpec((_BLK, D_), lambda i: (i, 0)),
                  pl.BlockSpec((16, 4), lambda i: (i, 0)),
                  pl.BlockSpec((D_, 4), lambda i: (0, 0))],
        out_specs=[pl.BlockSpec((_BLK, D_), lambda i: (i, 0)),
                   pl.BlockSpec((8, D_), lambda i: (0, 0))],
        out_shape=[jax.ShapeDtypeStruct((_ROWS, D_), jnp.float32),
                   jax.ShapeDtypeStruct((8, D_), jnp.float32)],
    )(gx, nx4, w1x4)


def _run_mlp_mid(y, scale, shift, w, b, cout):
    grid = _ROWS // _BLK
    cin = y.shape[1]
    return pl.pallas_call(
        _mlp_mid_body,
        grid=(grid,),
        in_specs=[pl.BlockSpec((_BLK, cin), lambda i: (i, 0)),
                  pl.BlockSpec((1, cin), lambda i: (0, 0)),
                  pl.BlockSpec((1, cin), lambda i: (0, 0)),
                  pl.BlockSpec((cout, cin), lambda i: (0, 0)),
                  pl.BlockSpec((1, cout), lambda i: (0, 0))],
        out_specs=[pl.BlockSpec((_BLK, cout), lambda i: (i, 0)),
                   pl.BlockSpec((8, cout), lambda i: (0, 0))],
        out_shape=[jax.ShapeDtypeStruct((_ROWS, cout), jnp.float32),
                   jax.ShapeDtypeStruct((8, cout), jnp.float32)],
    )(y, scale.reshape(1, cin), shift.reshape(1, cin), w, b.reshape(1, cout))


def _run_mlp_out(y3, scale, shift):
    grid = _ROWS // 1024
    return pl.pallas_call(
        _mlp_out_body,
        grid=(grid,),
        in_specs=[pl.BlockSpec((1024, C3), lambda i: (i, 0)),
                  pl.BlockSpec((1, C3), lambda i: (0, 0)),
                  pl.BlockSpec((1, C3), lambda i: (0, 0))],
        out_specs=pl.BlockSpec((32, C3), lambda i: (i, 0)),
        out_shape=jax.ShapeDtypeStruct((B_ * NPT, C3), jnp.float32),
    )(y3, scale.reshape(1, C3), shift.reshape(1, C3))


def _bn_fold(s, g, bt):
    rn = jnp.float32(_ROWS)
    mean = s[0] / rn
    var = s[1] / rn - mean * mean
    inv = 1.0 / jnp.sqrt(var + 1e-5)
    scale = g * inv
    shift = bt - mean * scale
    return scale, shift


def kernel(xyz, points, seed_inds, W1, b1, g1, bt1, W2, b2, g2, bt2,
           W3, b3, g3, bt3):
    cent, coord = _run_fps(xyz)
    fpsi = cent[:, :, 0]
    nxyz = coord[:, :, :4].reshape(B_, NPT * 4)

    ptst = jnp.transpose(points, (0, 2, 1))
    xyzt = jnp.transpose(xyz, (0, 2, 1))
    tb = jnp.concatenate(
        [ptst, xyzt, jnp.zeros((B_, N_, RW - D_ - 3), jnp.float32)],
        axis=2).reshape(B_ * N_, RW)
    w1e = jnp.concatenate(
        [W1[:, 3:], W1[:, :3], jnp.zeros((D_, RW - D_ - 3), jnp.float32)], axis=1)
    t1 = _run_t1(tb, w1e, b1)

    gx, seedo = _run_sc(xyz.reshape(-1), nxyz.reshape(-1), fpsi.reshape(-1),
                        seed_inds.reshape(-1), t1)
    seedo = seedo.reshape(B_, NPT)

    nx4 = coord[:, :, :4].reshape(B_ * NPT, 4)
    w1x4 = jnp.pad(W1[:, :3], ((0, 0), (0, 1)))
    y1, s1 = _run_mlp1(gx, nx4, w1x4)
    sc1, sh1 = _bn_fold(s1, g1, bt1)
    y2, s2 = _run_mlp_mid(y1, sc1, sh1, W2, b2, D_)
    sc2, sh2 = _bn_fold(s2, g2, bt2)
    y3, s3 = _run_mlp_mid(y2, sc2, sh2, W3, b3, C3)
    sc3, sh3 = _bn_fold(s3, g3, bt3)
    xo = _run_mlp_out(y3, sc3, sh3)

    new_xyz = jnp.transpose(coord[:, :, :3], (0, 2, 1))
    x_out = jnp.transpose(xo.reshape(B_, NPT, C3), (0, 2, 1))
    return (new_xyz, x_out, seedo)
